# baseline (speedup 1.0000x reference)
import functools

import jax
import jax.numpy as jnp
from jax import lax
from jax.experimental import pallas as pl
import jax.experimental.pallas.tpu as pltpu
from jax.experimental.pallas import tpu_sc as plsc

_B, _H, _KV, _Q, _DH = 8, 16, 2048, 16, 128
_BH = _B * _H
_SPLIT = 64

_RC = 2
_NBUF = 8
_L = 4

_NC, _NS = 2, 16
_NW = _NC * _NS
_SRW = (_BH - _SPLIT) // _NW
_CH = 256
_NCH = _KV // _CH
_TW = _BH // _NW


def _tc_key(pk_ref, ks_ref, ok_ref, kbuf, kin, kout):
    n = _BH // _RC

    def in_copies(i, s):
        rows = pl.ds(i * _RC, _RC)
        return [
            pltpu.make_async_copy(pk_ref.at[rows], kbuf.at[s, :, pl.ds(0, _KV)], kin.at[s]),
            pltpu.make_async_copy(ks_ref.at[rows], kbuf.at[s, :, pl.ds(_KV, _Q)], kin.at[s]),
        ]

    def out_copies(i, s):
        rows = pl.ds(i * _RC, _RC)
        return [pltpu.make_async_copy(kbuf.at[s], ok_ref.at[rows], kout.at[s])]

    for j in range(_L):
        for c in in_copies(j, j % _NBUF):
            c.start()
    for i in range(n):
        s = i % _NBUF
        nxt = i + _L
        if nxt < n:
            if nxt - _NBUF >= 0:
                for c in out_copies(nxt - _NBUF, nxt % _NBUF):
                    c.wait()
            for c in in_copies(nxt, nxt % _NBUF):
                c.start()
        for c in in_copies(i, s):
            c.wait()
        for c in out_copies(i, s):
            c.start()
    for j in range(n - _NBUF, n):
        for c in out_copies(j, j % _NBUF):
            c.wait()


def _tc_value_head(pv_ref, ovp_ref, ov_ref, vbuf, vin, vout):
    n = _SPLIT // _RC

    def in_copies(i, s):
        rows = pl.ds(i * _RC, _RC)
        return [pltpu.make_async_copy(pv_ref.at[rows], vbuf.at[s], vin.at[s])]

    def out_copies(i, s):
        rows = pl.ds(i * _RC, _RC)
        return [pltpu.make_async_copy(vbuf.at[s], ov_ref.at[rows, pl.ds(0, _KV)], vout.at[s])]

    for j in range(_L):
        for c in in_copies(j, j % _NBUF):
            c.start()
    for i in range(n):
        s = i % _NBUF
        nxt = i + _L
        if nxt < n:
            if nxt - _NBUF >= 0:
                for c in out_copies(nxt - _NBUF, nxt % _NBUF):
                    c.wait()
            for c in in_copies(nxt, nxt % _NBUF):
                c.start()
        for c in in_copies(i, s):
            c.wait()
        for c in out_copies(i, s):
            c.start()
    for j in range(n - _NBUF, n):
        for c in out_copies(j, j % _NBUF):
            c.wait()


def _sc_value_tail(pv, vs, ov, vbuf, sbuf):
    wid = lax.axis_index("s") * _NC + lax.axis_index("c")
    base = _SPLIT + wid * _SRW
    for j in range(_SRW):
        row = base + j
        for c in range(_NCH):
            sl = pl.ds(c * _CH, _CH)
            pltpu.sync_copy(pv.at[row, sl], vbuf)
            pltpu.sync_copy(vbuf, ov.at[row, sl])
    tail = pl.ds(_KV, _Q)
    for t in range(_TW):
        row = wid * _TW + t
        pltpu.sync_copy(vs.at[row], sbuf)
        pltpu.sync_copy(sbuf, ov.at[row, tail])


def kernel(past_key, past_value, key_states, value_states, layer_idx):
    pk = past_key.reshape(_BH, _KV, _DH)
    pv = past_value.reshape(_BH, _KV, _DH)
    ks = key_states.reshape(_BH, _Q, _DH)
    vs = value_states.reshape(_BH, _Q, _DH)

    out_t = jax.ShapeDtypeStruct((_BH, _KV + _Q, _DH), jnp.float32)
    hbm_spec = pl.BlockSpec(memory_space=pltpu.MemorySpace.HBM)

    sc_mesh = plsc.VectorSubcoreMesh(
        core_axis_name="c", subcore_axis_name="s",
        num_cores=_NC, num_subcores=_NS)
    sc_value = functools.partial(
        pl.kernel, mesh=sc_mesh,
        out_type=out_t,
        scratch_types=[
            pltpu.VMEM((_CH, _DH), jnp.float32),
            pltpu.VMEM((_Q, _DH), jnp.float32),
        ],
    )(_sc_value_tail)
    ovp = sc_value(pv, vs)

    ok = pl.pallas_call(
        _tc_key,
        in_specs=[hbm_spec, hbm_spec],
        out_specs=hbm_spec,
        out_shape=out_t,
        scratch_shapes=[
            pltpu.MemorySpace.VMEM((_NBUF, _RC, _KV + _Q, _DH), jnp.float32),
            pltpu.SemaphoreType.DMA((_NBUF,)),
            pltpu.SemaphoreType.DMA((_NBUF,)),
        ],
    )(pk, ks)

    ov = pl.pallas_call(
        _tc_value_head,
        in_specs=[hbm_spec, hbm_spec],
        out_specs=hbm_spec,
        out_shape=out_t,
        input_output_aliases={1: 0},
        scratch_shapes=[
            pltpu.MemorySpace.VMEM((_NBUF, _RC, _KV, _DH), jnp.float32),
            pltpu.SemaphoreType.DMA((_NBUF,)),
            pltpu.SemaphoreType.DMA((_NBUF,)),
        ],
    )(pv, ovp)

    ok = ok.reshape(_B, _H, _KV + _Q, _DH)
    ov = ov.reshape(_B, _H, _KV + _Q, _DH)
    return (ok, ov)


# speedup vs baseline: 1.1150x; 1.1150x over previous
import jax
import jax.numpy as jnp
from jax.experimental import pallas as pl
import jax.experimental.pallas.tpu as pltpu

_B, _H, _KV, _Q, _DH = 8, 16, 2048, 16, 128
_BH = _B * _H
_RC = 2
_N = _BH // _RC
_NBUF = 10
_L = 5


def _dma_pipeline(pk_ref, pv_ref, ks_ref, vs_ref, ok_ref, ov_ref,
                  kbuf, vbuf, kin, kout, vin, vout):
    def in_copies(i, s):
        rows = pl.ds(i * _RC, _RC)
        return [
            pltpu.make_async_copy(pk_ref.at[rows], kbuf.at[s, :, pl.ds(0, _KV)], kin.at[s]),
            pltpu.make_async_copy(ks_ref.at[rows], kbuf.at[s, :, pl.ds(_KV, _Q)], kin.at[s]),
            pltpu.make_async_copy(pv_ref.at[rows], vbuf.at[s, :, pl.ds(0, _KV)], vin.at[s]),
            pltpu.make_async_copy(vs_ref.at[rows], vbuf.at[s, :, pl.ds(_KV, _Q)], vin.at[s]),
        ]

    def out_copies(i, s):
        rows = pl.ds(i * _RC, _RC)
        return [
            pltpu.make_async_copy(kbuf.at[s], ok_ref.at[rows], kout.at[s]),
            pltpu.make_async_copy(vbuf.at[s], ov_ref.at[rows], vout.at[s]),
        ]

    for j in range(_L):
        for c in in_copies(j, j % _NBUF):
            c.start()
    for i in range(_N):
        s = i % _NBUF
        nxt = i + _L
        if nxt < _N:
            if nxt - _NBUF >= 0:
                for c in out_copies(nxt - _NBUF, nxt % _NBUF):
                    c.wait()
            for c in in_copies(nxt, nxt % _NBUF):
                c.start()
        for c in in_copies(i, s):
            c.wait()
        for c in out_copies(i, s):
            c.start()
    for j in range(_N - _NBUF, _N):
        for c in out_copies(j, j % _NBUF):
            c.wait()


def kernel(past_key, past_value, key_states, value_states, layer_idx):
    pk = past_key.reshape(_BH, _KV, _DH)
    pv = past_value.reshape(_BH, _KV, _DH)
    ks = key_states.reshape(_BH, _Q, _DH)
    vs = value_states.reshape(_BH, _Q, _DH)

    hbm_spec = pl.BlockSpec(memory_space=pltpu.MemorySpace.HBM)
    out_shape = jax.ShapeDtypeStruct((_BH, _KV + _Q, _DH), jnp.float32)

    ok, ov = pl.pallas_call(
        _dma_pipeline,
        in_specs=[hbm_spec] * 4,
        out_specs=[hbm_spec, hbm_spec],
        out_shape=[out_shape, out_shape],
        scratch_shapes=[
            pltpu.MemorySpace.VMEM((_NBUF, _RC, _KV + _Q, _DH), jnp.float32),
            pltpu.MemorySpace.VMEM((_NBUF, _RC, _KV + _Q, _DH), jnp.float32),
            pltpu.SemaphoreType.DMA((_NBUF,)),
            pltpu.SemaphoreType.DMA((_NBUF,)),
            pltpu.SemaphoreType.DMA((_NBUF,)),
            pltpu.SemaphoreType.DMA((_NBUF,)),
        ],
    )(pk, pv, ks, vs)

    ok = ok.reshape(_B, _H, _KV + _Q, _DH)
    ov = ov.reshape(_B, _H, _KV + _Q, _DH)
    return (ok, ov)


# speedup vs baseline: 1.1157x; 1.0006x over previous
import jax
import jax.numpy as jnp
from jax.experimental import pallas as pl
import jax.experimental.pallas.tpu as pltpu

_B, _H, _KV, _Q, _DH = 8, 16, 2048, 16, 128
_BH = _B * _H
_RC = 4
_N = _BH // _RC
_NBUF = 5
_L = 2


def _dma_pipeline(pk_ref, pv_ref, ks_ref, vs_ref, ok_ref, ov_ref,
                  kbuf, vbuf, kin, kout, vin, vout):
    def in_copies(i, s):
        rows = pl.ds(i * _RC, _RC)
        return [
            pltpu.make_async_copy(pk_ref.at[rows], kbuf.at[s, :, pl.ds(0, _KV)], kin.at[s]),
            pltpu.make_async_copy(ks_ref.at[rows], kbuf.at[s, :, pl.ds(_KV, _Q)], kin.at[s]),
            pltpu.make_async_copy(pv_ref.at[rows], vbuf.at[s, :, pl.ds(0, _KV)], vin.at[s]),
            pltpu.make_async_copy(vs_ref.at[rows], vbuf.at[s, :, pl.ds(_KV, _Q)], vin.at[s]),
        ]

    def out_copies(i, s):
        rows = pl.ds(i * _RC, _RC)
        return [
            pltpu.make_async_copy(kbuf.at[s], ok_ref.at[rows], kout.at[s]),
            pltpu.make_async_copy(vbuf.at[s], ov_ref.at[rows], vout.at[s]),
        ]

    for j in range(_L):
        for c in in_copies(j, j % _NBUF):
            c.start()
    for i in range(_N):
        s = i % _NBUF
        nxt = i + _L
        if nxt < _N:
            if nxt - _NBUF >= 0:
                for c in out_copies(nxt - _NBUF, nxt % _NBUF):
                    c.wait()
            for c in in_copies(nxt, nxt % _NBUF):
                c.start()
        for c in in_copies(i, s):
            c.wait()
        for c in out_copies(i, s):
            c.start()
    for j in range(_N - _NBUF, _N):
        for c in out_copies(j, j % _NBUF):
            c.wait()


def kernel(past_key, past_value, key_states, value_states, layer_idx):
    pk = past_key.reshape(_BH, _KV, _DH)
    pv = past_value.reshape(_BH, _KV, _DH)
    ks = key_states.reshape(_BH, _Q, _DH)
    vs = value_states.reshape(_BH, _Q, _DH)

    hbm_spec = pl.BlockSpec(memory_space=pltpu.MemorySpace.HBM)
    out_shape = jax.ShapeDtypeStruct((_BH, _KV + _Q, _DH), jnp.float32)

    ok, ov = pl.pallas_call(
        _dma_pipeline,
        in_specs=[hbm_spec] * 4,
        out_specs=[hbm_spec, hbm_spec],
        out_shape=[out_shape, out_shape],
        scratch_shapes=[
            pltpu.MemorySpace.VMEM((_NBUF, _RC, _KV + _Q, _DH), jnp.float32),
            pltpu.MemorySpace.VMEM((_NBUF, _RC, _KV + _Q, _DH), jnp.float32),
            pltpu.SemaphoreType.DMA((_NBUF,)),
            pltpu.SemaphoreType.DMA((_NBUF,)),
            pltpu.SemaphoreType.DMA((_NBUF,)),
            pltpu.SemaphoreType.DMA((_NBUF,)),
        ],
    )(pk, pv, ks, vs)

    ok = ok.reshape(_B, _H, _KV + _Q, _DH)
    ov = ov.reshape(_B, _H, _KV + _Q, _DH)
    return (ok, ov)
